# baseline (speedup 1.0000x reference)
import jax
import jax.numpy as jnp
from jax.experimental import pallas as pl
from jax.experimental.pallas import tpu as pltpu

_N = 4096
_D = 256
_EPS = 1e-5
_BM = 512
_NB = _N // _BM
_SLAB = 256
_CW = 1024


def _affine(sum_row, sq_row, g, b):
    mean = sum_row * (1.0 / _N)
    var = sq_row * (1.0 / _N) - mean * mean
    a = g * jax.lax.rsqrt(var + _EPS)
    c = b - mean * a
    return a, c


def _fused_kernel(
    adj_ref, x_ref, wp_ref, wf2_ref, bn_ref, gbs1_ref, gbf2_ref,
    f2_ref, s2_ref,
    adj_scr, s_scr, u2_scr, u3_scr, u4_scr, s1n_scr, abuf, s2buf,
    acc_sum, acc_sq, asem, wsem,
):
    s = pl.program_id(0)
    i = pl.program_id(1)
    f32 = jnp.float32
    bf16 = jnp.bfloat16
    rows = pl.ds(i * _BM, _BM)

    @pl.when((s == 0) & (i == 0))
    def _():
        xb = x_ref[...].astype(bf16)
        sup = jnp.dot(xb, wp_ref[0:256, :], preferred_element_type=f32)
        s_scr[:, 0:128] = sup.astype(bf16)

    @pl.when((s == 2) & (i == 0))
    def _():
        a, c = _affine(acc_sum[:, 0:128], acc_sq[:, 0:128],
                       bn_ref[0:1, :], bn_ref[1:2, :])
        zn = (u2_scr[...] * a + c).astype(bf16)
        s_scr[:, 0:128] = jnp.dot(
            zn, wp_ref[384:512, :], preferred_element_type=f32).astype(bf16)
        s_scr[:, 128:256] = jnp.dot(
            zn, wp_ref[512:640, :], preferred_element_type=f32).astype(bf16)

    @pl.when((s == 3) & (i == 0))
    def _():
        a_s, c_s = _affine(acc_sum[:, 128:256], acc_sq[:, 128:256],
                           gbs1_ref[0:1, :], gbs1_ref[1:2, :])
        s1n_scr[...] = (u3_scr[:, 128:256].astype(f32) * a_s
                        + c_s).astype(bf16)
        a_f, c_f = _affine(acc_sum[:, 0:128], acc_sq[:, 0:128],
                           bn_ref[2:3, :], bn_ref[3:4, :])
        f1n = (u3_scr[:, 0:128].astype(f32) * a_f + c_f).astype(bf16)
        s_scr[...] = jnp.dot(
            f1n, wf2_ref[...], preferred_element_type=f32).astype(bf16)

    @pl.when((i == 0) & (s < 4))
    def _():
        acc_sum[...] = jnp.zeros_like(acc_sum)
        acc_sq[...] = jnp.zeros_like(acc_sq)

    @pl.when(s == 0)
    def _():
        @pl.when(i == 0)
        def _():
            for b in (0, 1):
                pltpu.make_async_copy(
                    adj_ref.at[pl.ds(b * _SLAB, _SLAB), :],
                    abuf.at[b], asem.at[b]).start()
        for b in (0, 1):
            k = 2 * i + b
            pltpu.make_async_copy(
                adj_ref.at[pl.ds(k * _SLAB, _SLAB), :],
                abuf.at[b], asem.at[b]).wait()
            adj_scr[pl.ds(k * _SLAB, _SLAB), :] = abuf[b].astype(bf16)

            @pl.when(i < _NB - 1)
            def _():
                pltpu.make_async_copy(
                    adj_ref.at[pl.ds((k + 2) * _SLAB, _SLAB), :],
                    abuf.at[b], asem.at[b]).start()
        u = jnp.maximum(jnp.dot(adj_scr[rows, :], s_scr[:, 0:128],
                                preferred_element_type=f32), 0.0)
        s_scr[rows, 128:256] = jnp.dot(
            u.astype(bf16), wp_ref[256:384, :],
            preferred_element_type=f32).astype(bf16)

    @pl.when(s == 1)
    def _():
        u = jnp.maximum(jnp.dot(adj_scr[rows, :], s_scr[:, 128:256],
                                preferred_element_type=f32), 0.0)
        u2_scr[rows, :] = u
        acc_sum[:, 0:128] += jnp.sum(u, axis=0, keepdims=True)
        acc_sq[:, 0:128] += jnp.sum(u * u, axis=0, keepdims=True)

    @pl.when(s == 2)
    def _():
        u = jnp.maximum(jnp.dot(adj_scr[rows, :], s_scr[...],
                                preferred_element_type=f32), 0.0)
        u3_scr[rows, :] = u.astype(bf16)
        acc_sum[...] += jnp.sum(u, axis=0, keepdims=True)
        acc_sq[...] += jnp.sum(u * u, axis=0, keepdims=True)

    @pl.when(s == 3)
    def _():
        u = jnp.maximum(jnp.dot(adj_scr[rows, :], s_scr[...],
                                preferred_element_type=f32), 0.0)
        u4_scr[rows, :] = u.astype(bf16)
        acc_sum[...] += jnp.sum(u, axis=0, keepdims=True)
        acc_sq[...] += jnp.sum(u * u, axis=0, keepdims=True)
        blk = s1n_scr[rows, :]
        for j in range(4):
            b = j % 2

            def _wait(b=b, j=j):
                pltpu.make_async_copy(
                    s2buf.at[b],
                    s2_ref.at[rows, pl.ds(j * _CW, _CW)],
                    wsem.at[b]).wait()
            if j < 2:
                pl.when(i > 0)(_wait)
            else:
                _wait()
            s2buf[b] = jax.lax.dot_general(
                blk, s1n_scr[pl.ds(j * _CW, _CW), :],
                (((1,), (1,)), ((), ())), preferred_element_type=f32)
            pltpu.make_async_copy(
                s2buf.at[b],
                s2_ref.at[rows, pl.ds(j * _CW, _CW)],
                wsem.at[b]).start()

    @pl.when(s == 4)
    def _():
        @pl.when(i == 0)
        def _():
            for b in (0, 1):
                pltpu.make_async_copy(
                    s2buf.at[b],
                    s2_ref.at[pl.ds((_NB - 1) * _BM, _BM),
                              pl.ds((2 + b) * _CW, _CW)],
                    wsem.at[b]).wait()
        a4, c4 = _affine(acc_sum[...], acc_sq[...], gbf2_ref[0:1, :],
                         gbf2_ref[1:2, :])
        f2_ref[...] = u4_scr[rows, :].astype(f32) * a4 + c4


def kernel(x, adj, W_enc1, W_enc2, bn_enc2_g, bn_enc2_b, W_fd1, bn_fd1_g,
           bn_fd1_b, W_fd2, bn_fd2_g, bn_fd2_b, W_sd1, bn_sd1_g, bn_sd1_b):
    f32 = jnp.float32
    bf16 = jnp.bfloat16
    wp = jnp.zeros((640, 128), bf16)
    wp = wp.at[0:256, :].set(W_enc1.astype(bf16))
    wp = wp.at[256:384, 0:64].set(W_enc2.astype(bf16))
    wp = wp.at[384:448, :].set(W_fd1.astype(bf16))
    wp = wp.at[512:576, :].set(W_sd1.astype(bf16))
    wf2b = W_fd2.astype(bf16)
    bn = jnp.zeros((4, 128), f32)
    bn = bn.at[0, :].set(jnp.ones((128,), f32).at[0:64].set(bn_enc2_g))
    bn = bn.at[1, 0:64].set(bn_enc2_b)
    bn = bn.at[2, :].set(bn_fd1_g)
    bn = bn.at[3, :].set(bn_fd1_b)
    gbs1 = jnp.stack([bn_sd1_g, bn_sd1_b])
    gbf2 = jnp.stack([bn_fd2_g, bn_fd2_b])

    full = lambda shape: pl.BlockSpec(shape, lambda s, i: (0, 0))
    f2, s2 = pl.pallas_call(
        _fused_kernel,
        grid=(5, _NB),
        in_specs=[
            pl.BlockSpec(memory_space=pl.ANY),
            full((_N, _D)), full((640, 128)), full((128, _D)),
            full((4, 128)), full((2, 128)), full((2, _D)),
        ],
        out_specs=[
            pl.BlockSpec((_BM, _D),
                         lambda s, i: (jnp.where(s == 4, i, 0), 0)),
            pl.BlockSpec(memory_space=pl.ANY),
        ],
        out_shape=[
            jax.ShapeDtypeStruct((_N, _D), f32),
            jax.ShapeDtypeStruct((_N, _N), f32),
        ],
        scratch_shapes=[
            pltpu.VMEM((_N, _N), bf16),
            pltpu.VMEM((_N, 256), bf16),
            pltpu.VMEM((_N, 128), f32),
            pltpu.VMEM((_N, 256), bf16),
            pltpu.VMEM((_N, 256), bf16),
            pltpu.VMEM((_N, 128), bf16),
            pltpu.VMEM((2, _SLAB, _N), f32),
            pltpu.VMEM((2, _BM, _CW), f32),
            pltpu.VMEM((1, 256), f32),
            pltpu.VMEM((1, 256), f32),
            pltpu.SemaphoreType.DMA((2,)),
            pltpu.SemaphoreType.DMA((2,)),
        ],
        compiler_params=pltpu.CompilerParams(
            dimension_semantics=("arbitrary", "arbitrary"),
            vmem_limit_bytes=100 * 1024 * 1024,
        ),
    )(adj, x, wp, wf2b, bn, gbs1, gbf2)

    return (f2, s2)


# speedup vs baseline: 1.1788x; 1.1788x over previous
import jax
import jax.numpy as jnp
from jax.experimental import pallas as pl
from jax.experimental.pallas import tpu as pltpu

_N = 4096
_D = 256
_EPS = 1e-5
_BM = 512
_NB = _N // _BM
_BM2 = 512
_NB2 = _N // _BM2


def _affine(sum_row, sq_row, g, b):
    mean = sum_row * (1.0 / _N)
    var = sq_row * (1.0 / _N) - mean * mean
    a = g * jax.lax.rsqrt(var + _EPS)
    c = b - mean * a
    return a, c


def _gcn_stack_kernel(
    adj_ref, x_ref, w1_ref, w2_ref, g2_ref, b2_ref, wf1_ref, gf1_ref,
    bf1_ref, wf2_ref, ws1_ref,
    u4_ref, u3s_ref, s1stats_ref, u4stats_ref,
    adj_scr, s_scr, h1_scr, u2_scr, u3_scr, acc_sum, acc_sq,
):
    s = pl.program_id(0)
    i = pl.program_id(1)
    f32 = jnp.float32
    bf16 = jnp.bfloat16

    @pl.when((s == 0) & (i == 0))
    def _():
        xb = x_ref[...].astype(bf16)
        sup = jnp.dot(xb, w1_ref[...], preferred_element_type=f32)
        s_scr[:, 0:128] = sup.astype(bf16)

    @pl.when((s == 1) & (i == 0))
    def _():
        h1 = h1_scr[...].astype(bf16)
        sup = jnp.dot(h1, w2_ref[...], preferred_element_type=f32)
        s_scr[:, 0:128] = sup.astype(bf16)

    @pl.when((s == 2) & (i == 0))
    def _():
        a, c = _affine(acc_sum[:, 0:128], acc_sq[:, 0:128],
                       g2_ref[...], b2_ref[...])
        zn = (u2_scr[...] * a + c).astype(bf16)
        s_scr[:, 0:128] = jnp.dot(
            zn, wf1_ref[...], preferred_element_type=f32).astype(bf16)
        s_scr[:, 128:256] = jnp.dot(
            zn, ws1_ref[...], preferred_element_type=f32).astype(bf16)

    @pl.when((s == 3) & (i == 0))
    def _():
        s1stats_ref[0:1, :] = acc_sum[:, 128:256]
        s1stats_ref[1:2, :] = acc_sq[:, 128:256]
        a, c = _affine(acc_sum[:, 0:128], acc_sq[:, 0:128],
                       gf1_ref[...], bf1_ref[...])
        f1n = (u3_scr[:, 0:128] * a + c).astype(bf16)
        s_scr[...] = jnp.dot(
            f1n, wf2_ref[...], preferred_element_type=f32).astype(bf16)

    @pl.when(i == 0)
    def _():
        acc_sum[...] = jnp.zeros_like(acc_sum)
        acc_sq[...] = jnp.zeros_like(acc_sq)

    rows = pl.ds(i * _BM, _BM)

    @pl.when(s == 0)
    def _():
        adj_scr[rows, :] = adj_ref[...].astype(bf16)
        u = jnp.maximum(jnp.dot(adj_scr[rows, :], s_scr[:, 0:128],
                                preferred_element_type=f32), 0.0)
        h1_scr[rows, :] = u

    @pl.when(s == 1)
    def _():
        u = jnp.maximum(jnp.dot(adj_scr[rows, :], s_scr[:, 0:128],
                                preferred_element_type=f32), 0.0)
        u2_scr[rows, :] = u
        acc_sum[:, 0:128] += jnp.sum(u, axis=0, keepdims=True)
        acc_sq[:, 0:128] += jnp.sum(u * u, axis=0, keepdims=True)

    @pl.when(s == 2)
    def _():
        u = jnp.maximum(jnp.dot(adj_scr[rows, :], s_scr[...],
                                preferred_element_type=f32), 0.0)
        u3_scr[rows, :] = u
        acc_sum[...] += jnp.sum(u, axis=0, keepdims=True)
        acc_sq[...] += jnp.sum(u * u, axis=0, keepdims=True)

    @pl.when(s == 3)
    def _():
        u = jnp.maximum(jnp.dot(adj_scr[rows, :], s_scr[...],
                                preferred_element_type=f32), 0.0)
        u4_ref[...] = u.astype(bf16)
        u3s_ref[...] = u3_scr[rows, 128:256].astype(bf16).astype(bf16)
        acc_sum[...] += jnp.sum(u, axis=0, keepdims=True)
        acc_sq[...] += jnp.sum(u * u, axis=0, keepdims=True)

    @pl.when((s == 3) & (i == _NB - 1))
    def _():
        u4stats_ref[0:1, :] = acc_sum[...]
        u4stats_ref[1:2, :] = acc_sq[...]


def _decode_kernel(u3s_ref, s1stats_ref, gs1_ref, bs1_ref, u4_ref,
                   u4stats_ref, gf2_ref, bf2_ref,
                   f2_ref, s2_ref, s1n_scr):
    i = pl.program_id(0)
    f32 = jnp.float32

    @pl.when(i == 0)
    def _():
        a, c = _affine(s1stats_ref[0:1, :], s1stats_ref[1:2, :],
                       gs1_ref[...], bs1_ref[...])
        s1n_scr[...] = (u3s_ref[...].astype(f32) * a + c).astype(jnp.bfloat16)

    blk = s1n_scr[pl.ds(i * _BM2, _BM2), :]
    s2_ref[...] = jax.lax.dot_general(
        blk, s1n_scr[...], (((1,), (1,)), ((), ())),
        preferred_element_type=f32)
    a4, c4 = _affine(u4stats_ref[0:1, :], u4stats_ref[1:2, :],
                     gf2_ref[...], bf2_ref[...])
    f2_ref[...] = u4_ref[...].astype(f32) * a4 + c4


def kernel(x, adj, W_enc1, W_enc2, bn_enc2_g, bn_enc2_b, W_fd1, bn_fd1_g,
           bn_fd1_b, W_fd2, bn_fd2_g, bn_fd2_b, W_sd1, bn_sd1_g, bn_sd1_b):
    f32 = jnp.float32
    bf16 = jnp.bfloat16
    w1b = W_enc1.astype(bf16)
    w2p = jnp.zeros((128, 128), bf16).at[:, 0:64].set(W_enc2.astype(bf16))
    g2p = jnp.ones((1, 128), f32).at[:, 0:64].set(bn_enc2_g)
    b2p = jnp.zeros((1, 128), f32).at[:, 0:64].set(bn_enc2_b)
    wf1p = jnp.zeros((128, 128), bf16).at[0:64, :].set(W_fd1.astype(bf16))
    ws1p = jnp.zeros((128, 128), bf16).at[0:64, :].set(W_sd1.astype(bf16))
    wf2b = W_fd2.astype(bf16)
    gf1 = bn_fd1_g.reshape(1, -1)
    bf1 = bn_fd1_b.reshape(1, -1)
    gf2 = bn_fd2_g.reshape(1, -1)
    bf2 = bn_fd2_b.reshape(1, -1)
    gs1 = bn_sd1_g.reshape(1, -1)
    bs1 = bn_sd1_b.reshape(1, -1)

    full = lambda shape: pl.BlockSpec(shape, lambda s, i: (0, 0))
    u4, u3s, s1stats, u4stats = pl.pallas_call(
        _gcn_stack_kernel,
        grid=(4, _NB),
        in_specs=[
            pl.BlockSpec((_BM, _N),
                         lambda s, i: (jnp.where(s == 0, i, _NB - 1), 0)),
            full((_N, _D)), full((_D, 128)), full((128, 128)),
            full((1, 128)), full((1, 128)), full((128, 128)),
            full((1, 128)), full((1, 128)), full((128, _D)),
            full((128, 128)),
        ],
        out_specs=[
            pl.BlockSpec((_BM, _D),
                         lambda s, i: (jnp.where(s == 3, i, 0), 0)),
            pl.BlockSpec((_BM, 128),
                         lambda s, i: (jnp.where(s == 3, i, 0), 0)),
            full((2, 128)), full((2, _D)),
        ],
        out_shape=[
            jax.ShapeDtypeStruct((_N, _D), jnp.bfloat16),
            jax.ShapeDtypeStruct((_N, 128), jnp.bfloat16),
            jax.ShapeDtypeStruct((2, 128), f32),
            jax.ShapeDtypeStruct((2, _D), f32),
        ],
        scratch_shapes=[
            pltpu.VMEM((_N, _N), jnp.bfloat16),
            pltpu.VMEM((_N, 256), jnp.bfloat16),
            pltpu.VMEM((_N, 128), f32),
            pltpu.VMEM((_N, 128), f32),
            pltpu.VMEM((_N, 256), f32),
            pltpu.VMEM((1, 256), f32),
            pltpu.VMEM((1, 256), f32),
        ],
        compiler_params=pltpu.CompilerParams(
            dimension_semantics=("arbitrary", "arbitrary"),
            vmem_limit_bytes=100 * 1024 * 1024,
        ),
    )(adj, x, w1b, w2p, g2p, b2p, wf1p, gf1, bf1, wf2b, ws1p)

    fullb = lambda shape: pl.BlockSpec(shape, lambda i: (0, 0))
    f2, s2 = pl.pallas_call(
        _decode_kernel,
        grid=(_NB2,),
        in_specs=[
            fullb((_N, 128)), fullb((2, 128)), fullb((1, 128)),
            fullb((1, 128)),
            pl.BlockSpec((_BM2, _D), lambda i: (i, 0)),
            fullb((2, _D)), fullb((1, _D)), fullb((1, _D)),
        ],
        out_specs=[
            pl.BlockSpec((_BM2, _D), lambda i: (i, 0)),
            pl.BlockSpec((_BM2, _N), lambda i: (i, 0)),
        ],
        out_shape=[
            jax.ShapeDtypeStruct((_N, _D), f32),
            jax.ShapeDtypeStruct((_N, _N), f32),
        ],
        scratch_shapes=[pltpu.VMEM((_N, 128), jnp.bfloat16)],
        compiler_params=pltpu.CompilerParams(
            dimension_semantics=("arbitrary",),
            vmem_limit_bytes=100 * 1024 * 1024,
        ),
    )(u3s, s1stats, gs1, bs1, u4, u4stats, gf2, bf2)

    return (f2, s2)


# speedup vs baseline: 1.2086x; 1.0253x over previous
import jax
import jax.numpy as jnp
from jax.experimental import pallas as pl
from jax.experimental.pallas import tpu as pltpu

_N = 4096
_D = 256
_EPS = 1e-5
_BM = 512
_NB = _N // _BM
_BM2 = 1024
_NB2 = _N // _BM2


def _affine(sum_row, sq_row, g, b):
    mean = sum_row * (1.0 / _N)
    var = sq_row * (1.0 / _N) - mean * mean
    a = g * jax.lax.rsqrt(var + _EPS)
    c = b - mean * a
    return a, c


def _gcn_stack_kernel(
    adj_ref, x_ref, w1_ref, w2_ref, g2_ref, b2_ref, wf1_ref, gf1_ref,
    bf1_ref, wf2_ref, ws1_ref,
    u4_ref, u3s_ref, s1stats_ref, u4stats_ref,
    adj_scr, s_scr, h1_scr, u2_scr, u3_scr, acc_sum, acc_sq,
):
    s = pl.program_id(0)
    i = pl.program_id(1)
    f32 = jnp.float32
    bf16 = jnp.bfloat16

    @pl.when((s == 0) & (i == 0))
    def _():
        sup = jnp.dot(x_ref[...], w1_ref[...], preferred_element_type=f32)
        s_scr[:, 0:128] = sup.astype(bf16)

    @pl.when((s == 1) & (i == 0))
    def _():
        h1 = h1_scr[...].astype(f32)
        sup = jnp.dot(h1, w2_ref[...], preferred_element_type=f32)
        s_scr[:, 0:128] = sup.astype(bf16)

    @pl.when((s == 2) & (i == 0))
    def _():
        a, c = _affine(acc_sum[:, 0:128], acc_sq[:, 0:128],
                       g2_ref[...], b2_ref[...])
        zn = u2_scr[...].astype(f32) * a + c
        s_scr[:, 0:128] = jnp.dot(
            zn, wf1_ref[...], preferred_element_type=f32).astype(bf16)
        s_scr[:, 128:256] = jnp.dot(
            zn, ws1_ref[...], preferred_element_type=f32).astype(bf16)

    @pl.when((s == 3) & (i == 0))
    def _():
        s1stats_ref[0:1, :] = acc_sum[:, 128:256]
        s1stats_ref[1:2, :] = acc_sq[:, 128:256]
        a, c = _affine(acc_sum[:, 0:128], acc_sq[:, 0:128],
                       gf1_ref[...], bf1_ref[...])
        f1n = u3_scr[:, 0:128].astype(f32) * a + c
        s_scr[...] = jnp.dot(
            f1n, wf2_ref[...], preferred_element_type=f32).astype(bf16)

    @pl.when(i == 0)
    def _():
        acc_sum[...] = jnp.zeros_like(acc_sum)
        acc_sq[...] = jnp.zeros_like(acc_sq)

    rows = pl.ds(i * _BM, _BM)

    @pl.when(s == 0)
    def _():
        adj_scr[rows, :] = adj_ref[...].astype(bf16)
        u = jnp.maximum(jnp.dot(adj_scr[rows, :], s_scr[:, 0:128],
                                preferred_element_type=f32), 0.0)
        h1_scr[rows, :] = u

    @pl.when(s == 1)
    def _():
        u = jnp.maximum(jnp.dot(adj_scr[rows, :], s_scr[:, 0:128],
                                preferred_element_type=f32), 0.0)
        u2_scr[rows, :] = u
        acc_sum[:, 0:128] += jnp.sum(u, axis=0, keepdims=True)
        acc_sq[:, 0:128] += jnp.sum(u * u, axis=0, keepdims=True)

    @pl.when(s == 2)
    def _():
        u = jnp.maximum(jnp.dot(adj_scr[rows, :], s_scr[...],
                                preferred_element_type=f32), 0.0)
        u3_scr[rows, :] = u
        acc_sum[...] += jnp.sum(u, axis=0, keepdims=True)
        acc_sq[...] += jnp.sum(u * u, axis=0, keepdims=True)

    @pl.when(s == 3)
    def _():
        u = jnp.maximum(jnp.dot(adj_scr[rows, :], s_scr[...],
                                preferred_element_type=f32), 0.0)
        u4_ref[...] = u.astype(bf16)
        u3s_ref[...] = u3_scr[rows, 128:256].astype(bf16).astype(bf16)
        acc_sum[...] += jnp.sum(u, axis=0, keepdims=True)
        acc_sq[...] += jnp.sum(u * u, axis=0, keepdims=True)

    @pl.when((s == 3) & (i == _NB - 1))
    def _():
        u4stats_ref[0:1, :] = acc_sum[...]
        u4stats_ref[1:2, :] = acc_sq[...]


def _decode_kernel(u3s_ref, s1stats_ref, gs1_ref, bs1_ref, u4_ref,
                   u4stats_ref, gf2_ref, bf2_ref,
                   f2_ref, s2_ref, s1n_scr):
    i = pl.program_id(0)
    f32 = jnp.float32

    @pl.when(i == 0)
    def _():
        a, c = _affine(s1stats_ref[0:1, :], s1stats_ref[1:2, :],
                       gs1_ref[...], bs1_ref[...])
        s1n_scr[...] = (u3s_ref[...].astype(f32) * a + c).astype(jnp.bfloat16)

    blk = s1n_scr[pl.ds(i * _BM2, _BM2), :]
    s2_ref[...] = jax.lax.dot_general(
        blk, s1n_scr[...], (((1,), (1,)), ((), ())),
        preferred_element_type=f32)
    a4, c4 = _affine(u4stats_ref[0:1, :], u4stats_ref[1:2, :],
                     gf2_ref[...], bf2_ref[...])
    f2_ref[...] = u4_ref[...].astype(f32) * a4 + c4


def kernel(x, adj, W_enc1, W_enc2, bn_enc2_g, bn_enc2_b, W_fd1, bn_fd1_g,
           bn_fd1_b, W_fd2, bn_fd2_g, bn_fd2_b, W_sd1, bn_sd1_g, bn_sd1_b):
    f32 = jnp.float32
    w2p = jnp.zeros((128, 128), f32).at[:, 0:64].set(W_enc2)
    g2p = jnp.ones((1, 128), f32).at[:, 0:64].set(bn_enc2_g)
    b2p = jnp.zeros((1, 128), f32).at[:, 0:64].set(bn_enc2_b)
    wf1p = jnp.zeros((128, 128), f32).at[0:64, :].set(W_fd1)
    ws1p = jnp.zeros((128, 128), f32).at[0:64, :].set(W_sd1)
    gf1 = bn_fd1_g.reshape(1, -1)
    bf1 = bn_fd1_b.reshape(1, -1)
    gf2 = bn_fd2_g.reshape(1, -1)
    bf2 = bn_fd2_b.reshape(1, -1)
    gs1 = bn_sd1_g.reshape(1, -1)
    bs1 = bn_sd1_b.reshape(1, -1)

    full = lambda shape: pl.BlockSpec(shape, lambda s, i: (0, 0))
    u4, u3s, s1stats, u4stats = pl.pallas_call(
        _gcn_stack_kernel,
        grid=(4, _NB),
        in_specs=[
            pl.BlockSpec((_BM, _N),
                         lambda s, i: (jnp.where(s == 0, i, _NB - 1), 0)),
            full((_N, _D)), full((_D, 128)), full((128, 128)),
            full((1, 128)), full((1, 128)), full((128, 128)),
            full((1, 128)), full((1, 128)), full((128, _D)),
            full((128, 128)),
        ],
        out_specs=[
            pl.BlockSpec((_BM, _D),
                         lambda s, i: (jnp.where(s == 3, i, 0), 0)),
            pl.BlockSpec((_BM, 128),
                         lambda s, i: (jnp.where(s == 3, i, 0), 0)),
            full((2, 128)), full((2, _D)),
        ],
        out_shape=[
            jax.ShapeDtypeStruct((_N, _D), jnp.bfloat16),
            jax.ShapeDtypeStruct((_N, 128), jnp.bfloat16),
            jax.ShapeDtypeStruct((2, 128), f32),
            jax.ShapeDtypeStruct((2, _D), f32),
        ],
        scratch_shapes=[
            pltpu.VMEM((_N, _N), jnp.bfloat16),
            pltpu.VMEM((_N, 256), jnp.bfloat16),
            pltpu.VMEM((_N, 128), f32),
            pltpu.VMEM((_N, 128), f32),
            pltpu.VMEM((_N, 256), f32),
            pltpu.VMEM((1, 256), f32),
            pltpu.VMEM((1, 256), f32),
        ],
        compiler_params=pltpu.CompilerParams(
            dimension_semantics=("arbitrary", "arbitrary"),
            vmem_limit_bytes=100 * 1024 * 1024,
        ),
    )(adj, x, W_enc1, w2p, g2p, b2p, wf1p, gf1, bf1, W_fd2, ws1p)

    fullb = lambda shape: pl.BlockSpec(shape, lambda i: (0, 0))
    f2, s2 = pl.pallas_call(
        _decode_kernel,
        grid=(_NB2,),
        in_specs=[
            fullb((_N, 128)), fullb((2, 128)), fullb((1, 128)),
            fullb((1, 128)),
            pl.BlockSpec((_BM2, _D), lambda i: (i, 0)),
            fullb((2, _D)), fullb((1, _D)), fullb((1, _D)),
        ],
        out_specs=[
            pl.BlockSpec((_BM2, _D), lambda i: (i, 0)),
            pl.BlockSpec((_BM2, _N), lambda i: (i, 0)),
        ],
        out_shape=[
            jax.ShapeDtypeStruct((_N, _D), f32),
            jax.ShapeDtypeStruct((_N, _N), f32),
        ],
        scratch_shapes=[pltpu.VMEM((_N, 128), jnp.bfloat16)],
        compiler_params=pltpu.CompilerParams(
            dimension_semantics=("arbitrary",),
            vmem_limit_bytes=100 * 1024 * 1024,
        ),
    )(u3s, s1stats, gs1, bs1, u4, u4stats, gf2, bf2)

    return (f2, s2)


# speedup vs baseline: 1.2159x; 1.0060x over previous
import jax
import jax.numpy as jnp
from jax.experimental import pallas as pl
from jax.experimental.pallas import tpu as pltpu

_N = 4096
_D = 256
_EPS = 1e-5
_BM = 512
_NB = _N // _BM
_BM2 = 512
_NB2 = _N // _BM2


def _affine(sum_row, sq_row, g, b):
    mean = sum_row * (1.0 / _N)
    var = sq_row * (1.0 / _N) - mean * mean
    a = g * jax.lax.rsqrt(var + _EPS)
    c = b - mean * a
    return a, c


def _gcn_stack_kernel(
    adj_ref, x_ref, w1_ref, w2_ref, g2_ref, b2_ref, wf1_ref, gf1_ref,
    bf1_ref, wf2_ref, ws1_ref,
    u4_ref, u3s_ref, s1stats_ref, u4stats_ref,
    adj_scr, s_scr, h1_scr, u2_scr, u3_scr, acc_sum, acc_sq,
):
    s = pl.program_id(0)
    i = pl.program_id(1)
    f32 = jnp.float32
    bf16 = jnp.bfloat16

    @pl.when((s == 0) & (i == 0))
    def _():
        sup = jnp.dot(x_ref[...], w1_ref[...], preferred_element_type=f32)
        s_scr[:, 0:128] = sup.astype(bf16)

    @pl.when((s == 1) & (i == 0))
    def _():
        h1 = h1_scr[...].astype(f32)
        sup = jnp.dot(h1, w2_ref[...], preferred_element_type=f32)
        s_scr[:, 0:128] = sup.astype(bf16)

    @pl.when((s == 2) & (i == 0))
    def _():
        a, c = _affine(acc_sum[:, 0:128], acc_sq[:, 0:128],
                       g2_ref[...], b2_ref[...])
        zn = u2_scr[...].astype(f32) * a + c
        s_scr[:, 0:128] = jnp.dot(
            zn, wf1_ref[...], preferred_element_type=f32).astype(bf16)
        s_scr[:, 128:256] = jnp.dot(
            zn, ws1_ref[...], preferred_element_type=f32).astype(bf16)

    @pl.when((s == 3) & (i == 0))
    def _():
        s1stats_ref[0:1, :] = acc_sum[:, 128:256]
        s1stats_ref[1:2, :] = acc_sq[:, 128:256]
        a, c = _affine(acc_sum[:, 0:128], acc_sq[:, 0:128],
                       gf1_ref[...], bf1_ref[...])
        f1n = u3_scr[:, 0:128].astype(f32) * a + c
        s_scr[...] = jnp.dot(
            f1n, wf2_ref[...], preferred_element_type=f32).astype(bf16)

    @pl.when(i == 0)
    def _():
        acc_sum[...] = jnp.zeros_like(acc_sum)
        acc_sq[...] = jnp.zeros_like(acc_sq)

    rows = pl.ds(i * _BM, _BM)

    @pl.when(s == 0)
    def _():
        adj_scr[rows, :] = adj_ref[...].astype(bf16)
        u = jnp.maximum(jnp.dot(adj_scr[rows, :], s_scr[:, 0:128],
                                preferred_element_type=f32), 0.0)
        h1_scr[rows, :] = u

    @pl.when(s == 1)
    def _():
        u = jnp.maximum(jnp.dot(adj_scr[rows, :], s_scr[:, 0:128],
                                preferred_element_type=f32), 0.0)
        u2_scr[rows, :] = u
        acc_sum[:, 0:128] += jnp.sum(u, axis=0, keepdims=True)
        acc_sq[:, 0:128] += jnp.sum(u * u, axis=0, keepdims=True)

    @pl.when(s == 2)
    def _():
        u = jnp.maximum(jnp.dot(adj_scr[rows, :], s_scr[...],
                                preferred_element_type=f32), 0.0)
        u3_scr[rows, :] = u
        acc_sum[...] += jnp.sum(u, axis=0, keepdims=True)
        acc_sq[...] += jnp.sum(u * u, axis=0, keepdims=True)

    @pl.when(s == 3)
    def _():
        u = jnp.maximum(jnp.dot(adj_scr[rows, :], s_scr[...],
                                preferred_element_type=f32), 0.0)
        u4_ref[...] = u.astype(bf16)
        u3s_ref[...] = u3_scr[rows, 128:256].astype(bf16).astype(bf16)
        acc_sum[...] += jnp.sum(u, axis=0, keepdims=True)
        acc_sq[...] += jnp.sum(u * u, axis=0, keepdims=True)

    @pl.when((s == 3) & (i == _NB - 1))
    def _():
        u4stats_ref[0:1, :] = acc_sum[...]
        u4stats_ref[1:2, :] = acc_sq[...]


def _decode_kernel(u3s_ref, s1stats_ref, gs1_ref, bs1_ref, u4_ref,
                   u4stats_ref, gf2_ref, bf2_ref,
                   f2_ref, s2_ref, s1n_scr):
    i = pl.program_id(0)
    f32 = jnp.float32

    @pl.when(i == 0)
    def _():
        a, c = _affine(s1stats_ref[0:1, :], s1stats_ref[1:2, :],
                       gs1_ref[...], bs1_ref[...])
        s1n_scr[...] = (u3s_ref[...].astype(f32) * a + c).astype(jnp.bfloat16)

    blk = s1n_scr[pl.ds(i * _BM2, _BM2), :]
    s2_ref[...] = jax.lax.dot_general(
        blk, s1n_scr[...], (((1,), (1,)), ((), ())),
        preferred_element_type=f32)
    a4, c4 = _affine(u4stats_ref[0:1, :], u4stats_ref[1:2, :],
                     gf2_ref[...], bf2_ref[...])
    f2_ref[...] = u4_ref[...].astype(f32) * a4 + c4


def kernel(x, adj, W_enc1, W_enc2, bn_enc2_g, bn_enc2_b, W_fd1, bn_fd1_g,
           bn_fd1_b, W_fd2, bn_fd2_g, bn_fd2_b, W_sd1, bn_sd1_g, bn_sd1_b):
    f32 = jnp.float32
    w2p = jnp.zeros((128, 128), f32).at[:, 0:64].set(W_enc2)
    g2p = jnp.ones((1, 128), f32).at[:, 0:64].set(bn_enc2_g)
    b2p = jnp.zeros((1, 128), f32).at[:, 0:64].set(bn_enc2_b)
    wf1p = jnp.zeros((128, 128), f32).at[0:64, :].set(W_fd1)
    ws1p = jnp.zeros((128, 128), f32).at[0:64, :].set(W_sd1)
    gf1 = bn_fd1_g.reshape(1, -1)
    bf1 = bn_fd1_b.reshape(1, -1)
    gf2 = bn_fd2_g.reshape(1, -1)
    bf2 = bn_fd2_b.reshape(1, -1)
    gs1 = bn_sd1_g.reshape(1, -1)
    bs1 = bn_sd1_b.reshape(1, -1)

    full = lambda shape: pl.BlockSpec(shape, lambda s, i: (0, 0))
    u4, u3s, s1stats, u4stats = pl.pallas_call(
        _gcn_stack_kernel,
        grid=(4, _NB),
        in_specs=[
            pl.BlockSpec((_BM, _N),
                         lambda s, i: (jnp.where(s == 0, i, _NB - 1), 0)),
            full((_N, _D)), full((_D, 128)), full((128, 128)),
            full((1, 128)), full((1, 128)), full((128, 128)),
            full((1, 128)), full((1, 128)), full((128, _D)),
            full((128, 128)),
        ],
        out_specs=[
            pl.BlockSpec((_BM, _D),
                         lambda s, i: (jnp.where(s == 3, i, 0), 0)),
            pl.BlockSpec((_BM, 128),
                         lambda s, i: (jnp.where(s == 3, i, 0), 0)),
            full((2, 128)), full((2, _D)),
        ],
        out_shape=[
            jax.ShapeDtypeStruct((_N, _D), jnp.bfloat16),
            jax.ShapeDtypeStruct((_N, 128), jnp.bfloat16),
            jax.ShapeDtypeStruct((2, 128), f32),
            jax.ShapeDtypeStruct((2, _D), f32),
        ],
        scratch_shapes=[
            pltpu.VMEM((_N, _N), jnp.bfloat16),
            pltpu.VMEM((_N, 256), jnp.bfloat16),
            pltpu.VMEM((_N, 128), f32),
            pltpu.VMEM((_N, 128), f32),
            pltpu.VMEM((_N, 256), f32),
            pltpu.VMEM((1, 256), f32),
            pltpu.VMEM((1, 256), f32),
        ],
        compiler_params=pltpu.CompilerParams(
            dimension_semantics=("arbitrary", "arbitrary"),
            vmem_limit_bytes=100 * 1024 * 1024,
        ),
    )(adj, x, W_enc1, w2p, g2p, b2p, wf1p, gf1, bf1, W_fd2, ws1p)

    fullb = lambda shape: pl.BlockSpec(shape, lambda i: (0, 0))
    f2, s2 = pl.pallas_call(
        _decode_kernel,
        grid=(_NB2,),
        in_specs=[
            fullb((_N, 128)), fullb((2, 128)), fullb((1, 128)),
            fullb((1, 128)),
            pl.BlockSpec((_BM2, _D), lambda i: (i, 0)),
            fullb((2, _D)), fullb((1, _D)), fullb((1, _D)),
        ],
        out_specs=[
            pl.BlockSpec((_BM2, _D), lambda i: (i, 0)),
            pl.BlockSpec((_BM2, _N), lambda i: (i, 0)),
        ],
        out_shape=[
            jax.ShapeDtypeStruct((_N, _D), f32),
            jax.ShapeDtypeStruct((_N, _N), f32),
        ],
        scratch_shapes=[pltpu.VMEM((_N, 128), jnp.bfloat16)],
        compiler_params=pltpu.CompilerParams(
            dimension_semantics=("arbitrary",),
            vmem_limit_bytes=100 * 1024 * 1024,
        ),
    )(u3s, s1stats, gs1, bs1, u4, u4stats, gf2, bf2)

    return (f2, s2)


# speedup vs baseline: 1.2248x; 1.0073x over previous
import jax
import jax.numpy as jnp
from jax.experimental import pallas as pl
from jax.experimental.pallas import tpu as pltpu

_N = 4096
_D = 256
_EPS = 1e-5
_BM = 512
_NB = _N // _BM
_BM2 = 512
_NB2 = _N // _BM2


def _affine(sum_row, sq_row, g, b):
    mean = sum_row * (1.0 / _N)
    var = sq_row * (1.0 / _N) - mean * mean
    a = g * jax.lax.rsqrt(var + _EPS)
    c = b - mean * a
    return a, c


def _gcn_stack_kernel(
    adj_ref, x_ref, w1_ref, w2_ref, g2_ref, b2_ref, wf1_ref, gf1_ref,
    bf1_ref, wf2_ref, ws1_ref,
    u4_ref, u3s_ref, s1stats_ref, u4stats_ref,
    adj_scr, s_scr, u2_scr, u3_scr, acc_sum, acc_sq,
):
    s = pl.program_id(0)
    i = pl.program_id(1)
    f32 = jnp.float32
    bf16 = jnp.bfloat16

    @pl.when((s == 0) & (i == 0))
    def _():
        sup = jnp.dot(x_ref[...], w1_ref[...], preferred_element_type=f32)
        s_scr[:, 0:128] = sup.astype(bf16)

    @pl.when((s == 2) & (i == 0))
    def _():
        a, c = _affine(acc_sum[:, 0:128], acc_sq[:, 0:128],
                       g2_ref[...], b2_ref[...])
        zn = u2_scr[...].astype(f32) * a + c
        s_scr[:, 0:128] = jnp.dot(
            zn, wf1_ref[...], preferred_element_type=f32).astype(bf16)
        s_scr[:, 128:256] = jnp.dot(
            zn, ws1_ref[...], preferred_element_type=f32).astype(bf16)

    @pl.when((s == 3) & (i == 0))
    def _():
        s1stats_ref[0:1, :] = acc_sum[:, 128:256]
        s1stats_ref[1:2, :] = acc_sq[:, 128:256]
        a, c = _affine(acc_sum[:, 0:128], acc_sq[:, 0:128],
                       gf1_ref[...], bf1_ref[...])
        f1n = u3_scr[:, 0:128].astype(f32) * a + c
        s_scr[...] = jnp.dot(
            f1n, wf2_ref[...], preferred_element_type=f32).astype(bf16)

    @pl.when(i == 0)
    def _():
        acc_sum[...] = jnp.zeros_like(acc_sum)
        acc_sq[...] = jnp.zeros_like(acc_sq)

    rows = pl.ds(i * _BM, _BM)

    @pl.when(s == 0)
    def _():
        adj_scr[rows, :] = adj_ref[...].astype(bf16)
        u = jnp.maximum(jnp.dot(adj_scr[rows, :], s_scr[:, 0:128],
                                preferred_element_type=f32), 0.0)
        s2rows = jnp.dot(u, w2_ref[...], preferred_element_type=f32)
        s_scr[rows, 128:256] = s2rows.astype(bf16)

    @pl.when(s == 1)
    def _():
        u = jnp.maximum(jnp.dot(adj_scr[rows, :], s_scr[:, 128:256],
                                preferred_element_type=f32), 0.0)
        u2_scr[rows, :] = u
        acc_sum[:, 0:128] += jnp.sum(u, axis=0, keepdims=True)
        acc_sq[:, 0:128] += jnp.sum(u * u, axis=0, keepdims=True)

    @pl.when(s == 2)
    def _():
        u = jnp.maximum(jnp.dot(adj_scr[rows, :], s_scr[...],
                                preferred_element_type=f32), 0.0)
        u3_scr[rows, :] = u
        acc_sum[...] += jnp.sum(u, axis=0, keepdims=True)
        acc_sq[...] += jnp.sum(u * u, axis=0, keepdims=True)

    @pl.when(s == 3)
    def _():
        u = jnp.maximum(jnp.dot(adj_scr[rows, :], s_scr[...],
                                preferred_element_type=f32), 0.0)
        u4_ref[...] = u.astype(bf16)
        u3s_ref[...] = u3_scr[rows, 128:256].astype(bf16).astype(bf16)
        acc_sum[...] += jnp.sum(u, axis=0, keepdims=True)
        acc_sq[...] += jnp.sum(u * u, axis=0, keepdims=True)

    @pl.when((s == 3) & (i == _NB - 1))
    def _():
        u4stats_ref[0:1, :] = acc_sum[...]
        u4stats_ref[1:2, :] = acc_sq[...]


def _decode_kernel(u3s_ref, s1stats_ref, gs1_ref, bs1_ref, u4_ref,
                   u4stats_ref, gf2_ref, bf2_ref,
                   f2_ref, s2_ref, s1n_scr):
    i = pl.program_id(0)
    f32 = jnp.float32

    @pl.when(i == 0)
    def _():
        a, c = _affine(s1stats_ref[0:1, :], s1stats_ref[1:2, :],
                       gs1_ref[...], bs1_ref[...])
        s1n_scr[...] = (u3s_ref[...].astype(f32) * a + c).astype(jnp.bfloat16)

    blk = s1n_scr[pl.ds(i * _BM2, _BM2), :]
    s2_ref[...] = jax.lax.dot_general(
        blk, s1n_scr[...], (((1,), (1,)), ((), ())),
        preferred_element_type=f32)
    a4, c4 = _affine(u4stats_ref[0:1, :], u4stats_ref[1:2, :],
                     gf2_ref[...], bf2_ref[...])
    f2_ref[...] = u4_ref[...].astype(f32) * a4 + c4


def kernel(x, adj, W_enc1, W_enc2, bn_enc2_g, bn_enc2_b, W_fd1, bn_fd1_g,
           bn_fd1_b, W_fd2, bn_fd2_g, bn_fd2_b, W_sd1, bn_sd1_g, bn_sd1_b):
    f32 = jnp.float32
    w2p = jnp.zeros((128, 128), f32).at[:, 0:64].set(W_enc2)
    g2p = jnp.ones((1, 128), f32).at[:, 0:64].set(bn_enc2_g)
    b2p = jnp.zeros((1, 128), f32).at[:, 0:64].set(bn_enc2_b)
    wf1p = jnp.zeros((128, 128), f32).at[0:64, :].set(W_fd1)
    ws1p = jnp.zeros((128, 128), f32).at[0:64, :].set(W_sd1)
    gf1 = bn_fd1_g.reshape(1, -1)
    bf1 = bn_fd1_b.reshape(1, -1)
    gf2 = bn_fd2_g.reshape(1, -1)
    bf2 = bn_fd2_b.reshape(1, -1)
    gs1 = bn_sd1_g.reshape(1, -1)
    bs1 = bn_sd1_b.reshape(1, -1)

    full = lambda shape: pl.BlockSpec(shape, lambda s, i: (0, 0))
    u4, u3s, s1stats, u4stats = pl.pallas_call(
        _gcn_stack_kernel,
        grid=(4, _NB),
        in_specs=[
            pl.BlockSpec((_BM, _N),
                         lambda s, i: (jnp.where(s == 0, i, _NB - 1), 0)),
            full((_N, _D)), full((_D, 128)), full((128, 128)),
            full((1, 128)), full((1, 128)), full((128, 128)),
            full((1, 128)), full((1, 128)), full((128, _D)),
            full((128, 128)),
        ],
        out_specs=[
            pl.BlockSpec((_BM, _D),
                         lambda s, i: (jnp.where(s == 3, i, 0), 0)),
            pl.BlockSpec((_BM, 128),
                         lambda s, i: (jnp.where(s == 3, i, 0), 0)),
            full((2, 128)), full((2, _D)),
        ],
        out_shape=[
            jax.ShapeDtypeStruct((_N, _D), jnp.bfloat16),
            jax.ShapeDtypeStruct((_N, 128), jnp.bfloat16),
            jax.ShapeDtypeStruct((2, 128), f32),
            jax.ShapeDtypeStruct((2, _D), f32),
        ],
        scratch_shapes=[
            pltpu.VMEM((_N, _N), jnp.bfloat16),
            pltpu.VMEM((_N, 256), jnp.bfloat16),
            pltpu.VMEM((_N, 128), f32),
            pltpu.VMEM((_N, 256), f32),
            pltpu.VMEM((1, 256), f32),
            pltpu.VMEM((1, 256), f32),
        ],
        compiler_params=pltpu.CompilerParams(
            dimension_semantics=("arbitrary", "arbitrary"),
            vmem_limit_bytes=100 * 1024 * 1024,
        ),
    )(adj, x, W_enc1, w2p, g2p, b2p, wf1p, gf1, bf1, W_fd2, ws1p)

    fullb = lambda shape: pl.BlockSpec(shape, lambda i: (0, 0))
    f2, s2 = pl.pallas_call(
        _decode_kernel,
        grid=(_NB2,),
        in_specs=[
            fullb((_N, 128)), fullb((2, 128)), fullb((1, 128)),
            fullb((1, 128)),
            pl.BlockSpec((_BM2, _D), lambda i: (i, 0)),
            fullb((2, _D)), fullb((1, _D)), fullb((1, _D)),
        ],
        out_specs=[
            pl.BlockSpec((_BM2, _D), lambda i: (i, 0)),
            pl.BlockSpec((_BM2, _N), lambda i: (i, 0)),
        ],
        out_shape=[
            jax.ShapeDtypeStruct((_N, _D), f32),
            jax.ShapeDtypeStruct((_N, _N), f32),
        ],
        scratch_shapes=[pltpu.VMEM((_N, 128), jnp.bfloat16)],
        compiler_params=pltpu.CompilerParams(
            dimension_semantics=("arbitrary",),
            vmem_limit_bytes=100 * 1024 * 1024,
        ),
    )(u3s, s1stats, gs1, bs1, u4, u4stats, gf2, bf2)

    return (f2, s2)
